# Initial kernel scaffold; baseline (speedup 1.0000x reference)
import jax
import jax.numpy as jnp
from jax.experimental import pallas as pl


def kernel(x, edge_index, embed):
    raise NotImplementedError("write your pallas kernel here")


# speedup vs baseline: 4.7039x; 4.7039x over previous
import functools

import jax
import jax.numpy as jnp
from jax import lax
from jax.experimental import pallas as pl
from jax.experimental.pallas import tpu as pltpu
from jax.experimental.pallas import tpu_sc as plsc

_MASK_RATIO = 0.5
_NEIGH_WEIGHT = 0.5

_NC = 2
_NS = 16
_NW = _NC * _NS
_CH = 128
_ACC = 10240
_RPT = _ACC // _NS


def _sc_body(nchunk, d,
             row_ref, col_ref, emb_ref,
             acc_out, deg_out,
             row_v, col_v, buf, ones_v, zrow, zvec, acc_s, dr_s, dc_s):
    cid = lax.axis_index("c")
    sid = lax.axis_index("s")
    tid = cid * _NS + sid

    z16 = jnp.zeros((16,), jnp.float32)
    o16 = jnp.ones((16,), jnp.float32)
    for i in range(16):
        for j in range(d // 16):
            zrow[i, pl.ds(j * 16, 16)] = z16
    for k in range(_RPT // 16):
        zvec[pl.ds(k * 16, 16)] = z16
    for k in range(_CH // 16):
        ones_v[pl.ds(k * 16, 16)] = o16

    base = sid * _RPT
    for b in range(_RPT // 16):
        pltpu.sync_copy(zrow, acc_s.at[pl.ds(base + b * 16, 16)])
    pltpu.sync_copy(zvec, dr_s.at[pl.ds(base, _RPT)])
    pltpu.sync_copy(zvec, dc_s.at[pl.ds(base, _RPT)])
    plsc.subcore_barrier()

    pltpu.sync_copy(row_ref.at[tid], row_v)
    pltpu.sync_copy(col_ref.at[tid], col_v)

    def body(i, carry):
        pltpu.sync_copy(emb_ref.at[col_v.at[i]], buf)
        pltpu.sync_copy(buf, acc_s.at[row_v.at[i]], add=True)
        pltpu.sync_copy(ones_v, dr_s.at[row_v.at[i]], add=True)
        pltpu.sync_copy(ones_v, dc_s.at[col_v.at[i]], add=True)
        return carry

    lax.fori_loop(0, nchunk, body, 0)
    plsc.subcore_barrier()

    pltpu.sync_copy(acc_s.at[pl.ds(base, _RPT)],
                    acc_out.at[pl.ds(cid * _ACC + base, _RPT)])
    pltpu.sync_copy(dr_s.at[pl.ds(base, _RPT)],
                    deg_out.at[pl.ds(cid * 2 * _ACC + base, _RPT)])
    pltpu.sync_copy(dc_s.at[pl.ds(base, _RPT)],
                    deg_out.at[pl.ds(cid * 2 * _ACC + _ACC + base, _RPT)])


def _tc_body(num_mask, num_edges, emb_ref, acc_ref, deg_ref, mvec_ref, wt_ref,
             out_ref):
    emb = emb_ref[...]
    ns = acc_ref[0] + acc_ref[1]
    deg = deg_ref[...]
    dr_raw = deg[0] + deg[2]
    dc = deg[1] + deg[3]
    mvec = mvec_ref[...]

    r = jnp.dot(emb, wt_ref[...], preferred_element_type=jnp.float32)
    nm = ns / jnp.maximum(dr_raw, 1.0)[:, None]
    dvec = r - nm
    recon_sum = jnp.sum(mvec * jnp.sum(dvec * dvec, axis=1))
    nrm = jnp.sum(emb * emb, axis=1)
    sq_sum = jnp.sum((dr_raw + dc) * nrm)
    dot_sum = jnp.sum(emb * ns)

    d = emb.shape[1]
    recon_loss = recon_sum / (num_mask * d)
    neigh_loss = (sq_sum - 2.0 * dot_sum) / num_edges
    total = recon_loss + _NEIGH_WEIGHT * neigh_loss
    out_ref[...] = total[None, None]


def kernel(x, edge_index, embed):
    n, d = embed.shape
    e = edge_index.shape[1]
    num_mask = max(1, int(_MASK_RATIO * n))

    rkey = jax.random.key(42)
    perm = jax.random.permutation(rkey, n)
    mask_idx = perm[:num_mask]
    mvec = jnp.zeros((n,), jnp.float32).at[mask_idx].set(1.0)
    w = jax.random.normal(jax.random.fold_in(rkey, 1), (x.shape[1], d),
                          dtype=jnp.float32) * 0.01
    wt = w.T

    nchunk = -(-e // (_NW * _CH))
    tot = _NW * nchunk * _CH
    junk = jnp.int32(n)
    row = edge_index[0].astype(jnp.int32)
    col = edge_index[1].astype(jnp.int32)
    pad = tot - e
    row_p = jnp.concatenate([row, jnp.full((pad,), junk)]).reshape(_NW, nchunk, _CH)
    col_p = jnp.concatenate([col, jnp.full((pad,), junk)]).reshape(_NW, nchunk, _CH)
    emb_pad = jnp.concatenate([embed, jnp.zeros((16, d), jnp.float32)], axis=0)

    mesh = plsc.VectorSubcoreMesh(core_axis_name="c", subcore_axis_name="s")
    sc_fn = pl.kernel(
        functools.partial(_sc_body, nchunk, d),
        out_type=[
            jax.ShapeDtypeStruct((_NC * _ACC, d), jnp.float32),
            jax.ShapeDtypeStruct((_NC * 2 * _ACC,), jnp.float32),
        ],
        mesh=mesh,
        scratch_types=[
            pltpu.VMEM((nchunk, _CH), jnp.int32),
            pltpu.VMEM((nchunk, _CH), jnp.int32),
            pltpu.VMEM((_CH, d), jnp.float32),
            pltpu.VMEM((_CH,), jnp.float32),
            pltpu.VMEM((16, d), jnp.float32),
            pltpu.VMEM((_RPT,), jnp.float32),
            pltpu.VMEM_SHARED((_ACC, d), jnp.float32),
            pltpu.VMEM_SHARED((_ACC,), jnp.float32),
            pltpu.VMEM_SHARED((_ACC,), jnp.float32),
        ],
    )
    acc_out, deg_out = sc_fn(row_p, col_p, emb_pad)

    acc_p = acc_out.reshape(_NC, _ACC, d)[:, :n, :]
    deg_p = deg_out.reshape(_NC * 2, _ACC)[:, :n]

    out = pl.pallas_call(
        functools.partial(_tc_body, num_mask, e),
        out_shape=jax.ShapeDtypeStruct((1, 1), jnp.float32),
    )(embed, acc_p, deg_p, mvec, wt)
    return out[0, 0]
